# Initial kernel scaffold; baseline (speedup 1.0000x reference)
import functools

import jax
import jax.numpy as jnp
from jax import lax
from jax.experimental import pallas as pl
from jax.experimental.pallas import tpu as pltpu
from jax.experimental.pallas import tpu_sc as plsc

N = 10000
E = 320000
H = 16
F_IN = 128

NC = 2
NS = 16
NW = NC * NS
EPT = E // NW
CH = 80
NCHK = EPT // CH
NPAD = 10240
RPT = NPAD // NS

_mesh = plsc.VectorSubcoreMesh(core_axis_name="c", subcore_axis_name="s")


@functools.partial(
    pl.kernel,
    out_type=jax.ShapeDtypeStruct((NC * NPAD,), jnp.float32),
    mesh=_mesh,
    scratch_types=[
        pltpu.VMEM((NCHK, CH), jnp.int32),
        pltpu.VMEM((CH,), jnp.float32),
        pltpu.VMEM((RPT,), jnp.float32),
        pltpu.VMEM_SHARED((NPAD,), jnp.float32),
    ],
)
def _deg_kernel(dst_hbm, out_hbm, idx_v, ones_v, zb_v, acc_sh):
    c = lax.axis_index("c")
    s = lax.axis_index("s")
    wid = s * NC + c
    pltpu.sync_copy(dst_hbm.at[wid], idx_v)
    for k in range(CH // 16):
        ones_v[pl.ds(16 * k, 16)] = jnp.ones((16,), jnp.float32)
    for k in range(RPT // 16):
        zb_v[pl.ds(16 * k, 16)] = jnp.zeros((16,), jnp.float32)
    pltpu.sync_copy(zb_v, acc_sh.at[pl.ds(s * RPT, RPT)])
    plsc.subcore_barrier()

    def body(j, carry):
        pltpu.sync_copy(ones_v, acc_sh.at[idx_v.at[j]], add=True)
        return carry

    lax.fori_loop(0, NCHK, body, 0)
    plsc.subcore_barrier()
    pltpu.sync_copy(acc_sh.at[pl.ds(s * RPT, RPT)],
                    out_hbm.at[pl.ds(c * NPAD + s * RPT, RPT)])


@functools.partial(
    pl.kernel,
    out_type=jax.ShapeDtypeStruct((NC * NPAD, H), jnp.float32),
    mesh=_mesh,
    scratch_types=[
        pltpu.VMEM((NCHK, CH), jnp.int32),
        pltpu.VMEM((NCHK, CH), jnp.int32),
        pltpu.VMEM((CH, H), jnp.float32),
        pltpu.VMEM((RPT, H), jnp.float32),
        pltpu.VMEM_SHARED((NPAD, H), jnp.float32),
        pltpu.SemaphoreType.DMA,
    ],
)
def _scat_kernel(g_hbm, src_hbm, dst_hbm, z_hbm, out_hbm,
                 src_v, dst_v, rows_v, zb_v, acc_sh, sem):
    c = lax.axis_index("c")
    s = lax.axis_index("s")
    wid = s * NC + c
    pltpu.sync_copy(src_hbm.at[wid], src_v)
    pltpu.sync_copy(dst_hbm.at[wid], dst_v)
    pltpu.sync_copy(z_hbm.at[pl.ds(s * RPT, RPT)], zb_v)
    pltpu.sync_copy(zb_v, acc_sh.at[pl.ds(s * RPT, RPT)])
    plsc.subcore_barrier()

    def body(j, carry):
        pltpu.async_copy(g_hbm.at[src_v.at[j]], rows_v, sem).wait()
        pltpu.sync_copy(rows_v, acc_sh.at[dst_v.at[j]], add=True)
        return carry

    lax.fori_loop(0, NCHK, body, 0)
    plsc.subcore_barrier()
    pltpu.sync_copy(acc_sh.at[pl.ds(s * RPT, RPT)],
                    out_hbm.at[pl.ds(c * NPAD + s * RPT, RPT)])


@functools.partial(
    pl.kernel,
    out_type=jax.ShapeDtypeStruct((E,), jnp.float32),
    mesh=_mesh,
    scratch_types=[
        pltpu.VMEM((NCHK, CH), jnp.int32),
        pltpu.VMEM((NCHK, CH), jnp.int32),
        pltpu.VMEM((CH,), jnp.float32),
        pltpu.VMEM((CH,), jnp.float32),
        pltpu.VMEM((CH,), jnp.float32),
        pltpu.SemaphoreType.DMA,
        pltpu.SemaphoreType.DMA,
    ],
)
def _edge_kernel(a_hbm, c_hbm, src_hbm, dst_hbm, out_hbm,
                 src_v, dst_v, va_v, vc_v, vo_v, sem_a, sem_c):
    c = lax.axis_index("c")
    s = lax.axis_index("s")
    wid = s * NC + c
    base = wid * EPT
    pltpu.sync_copy(src_hbm.at[wid], src_v)
    pltpu.sync_copy(dst_hbm.at[wid], dst_v)

    def body(j, carry):
        cp_a = pltpu.async_copy(a_hbm.at[src_v.at[j]], va_v, sem_a)
        cp_c = pltpu.async_copy(c_hbm.at[dst_v.at[j]], vc_v, sem_c)
        cp_a.wait()
        cp_c.wait()
        for k in range(CH // 16):
            sl = pl.ds(16 * k, 16)
            vo_v[sl] = va_v[sl] + vc_v[sl]
        pltpu.sync_copy(vo_v, out_hbm.at[pl.ds(base + j * CH, CH)])
        return carry

    lax.fori_loop(0, NCHK, body, 0)


def _tc_deg_body(degp_ref, dis_ref, invd_ref):
    deg = degp_ref[0, :] + degp_ref[1, :] + 1.0
    dis_ref[...] = lax.rsqrt(deg)
    invd_ref[...] = 1.0 / deg


_tc_deg = pl.pallas_call(
    _tc_deg_body,
    out_shape=[jax.ShapeDtypeStruct((NPAD,), jnp.float32),
               jax.ShapeDtypeStruct((NPAD,), jnp.float32)],
)


def _tc_a_body(x_ref, w1_ref, b1_ref, dis_ref, invd_ref, g0_ref, self1_ref):
    h0 = jnp.dot(x_ref[...], w1_ref[...], preferred_element_type=jnp.float32)
    g0_ref[...] = h0 * dis_ref[...]
    self1_ref[...] = h0 * invd_ref[...] + b1_ref[...]


_tc_a = pl.pallas_call(
    _tc_a_body,
    out_shape=[jax.ShapeDtypeStruct((N, H), jnp.float32),
               jax.ShapeDtypeStruct((N, H), jnp.float32)],
)


def _tc_b_body(s1_ref, self1_ref, dis_ref, invd_ref, w2_ref, b2_ref,
               g1_ref, self2_ref):
    ssum = s1_ref[0, :N, :] + s1_ref[1, :N, :]
    h1 = jnp.maximum(dis_ref[...] * ssum + self1_ref[...], 0.0)
    h1w = jnp.dot(h1, w2_ref[...], preferred_element_type=jnp.float32)
    g1_ref[...] = h1w * dis_ref[...]
    self2_ref[...] = h1w * invd_ref[...] + b2_ref[...]


_tc_b = pl.pallas_call(
    _tc_b_body,
    out_shape=[jax.ShapeDtypeStruct((N, H), jnp.float32),
               jax.ShapeDtypeStruct((N, H), jnp.float32)],
)


def _tc_c_body(s2_ref, self2_ref, dis_ref, wf_ref, bf_ref, a_ref, c_ref):
    ssum = s2_ref[0, :N, :] + s2_ref[1, :N, :]
    h2 = dis_ref[...] * ssum + self2_ref[...]
    a_ref[...] = jnp.dot(h2, wf_ref[:H, :],
                         preferred_element_type=jnp.float32) + bf_ref[...]
    c_ref[...] = jnp.dot(h2, wf_ref[H:, :],
                         preferred_element_type=jnp.float32)


_tc_c = pl.pallas_call(
    _tc_c_body,
    out_shape=[jax.ShapeDtypeStruct((N, 1), jnp.float32),
               jax.ShapeDtypeStruct((N, 1), jnp.float32)],
)


def kernel(x, edge_index, W1, b1, W2, b2, Wf, bf):
    src3 = edge_index[0].reshape(NW, NCHK, CH)
    dst3 = edge_index[1].reshape(NW, NCHK, CH)
    z16 = jnp.zeros((NPAD, H), jnp.float32)

    degp = _deg_kernel(dst3).reshape(NC, NPAD)
    dis_pad, invd_pad = _tc_deg(degp)
    dis = dis_pad[:N].reshape(N, 1)
    invd = invd_pad[:N].reshape(N, 1)

    g0, self1 = _tc_a(x, W1, b1, dis, invd)
    s1 = _scat_kernel(g0, src3, dst3, z16).reshape(NC, NPAD, H)
    g1, self2 = _tc_b(s1, self1, dis, invd, W2, b2)
    s2 = _scat_kernel(g1, src3, dst3, z16).reshape(NC, NPAD, H)
    a2, c2 = _tc_c(s2, self2, dis, Wf, bf)

    return _edge_kernel(a2.reshape(N), c2.reshape(N), src3, dst3)


# speedup vs baseline: 23.9374x; 23.9374x over previous
import functools

import jax
import jax.numpy as jnp
from jax import lax
from jax.experimental import pallas as pl
from jax.experimental.pallas import tpu as pltpu
from jax.experimental.pallas import tpu_sc as plsc

N = 10000
E = 320000
H = 16
F_IN = 128

NC = 2
NS = 16
NW = NC * NS
EPT = E // NW
CH = 80
NCHK = EPT // CH
NPAD = 10240
RPT = NPAD // NS

_mesh = plsc.VectorSubcoreMesh(core_axis_name="c", subcore_axis_name="s")


@functools.partial(
    pl.kernel,
    out_type=jax.ShapeDtypeStruct((NC * NPAD,), jnp.float32),
    mesh=_mesh,
    compiler_params=pltpu.CompilerParams(use_tc_tiling_on_sc=False),
    scratch_types=[
        pltpu.VMEM((NCHK, CH), jnp.int32),
        pltpu.VMEM((CH,), jnp.float32),
        pltpu.VMEM((RPT,), jnp.float32),
        pltpu.VMEM_SHARED((NPAD,), jnp.float32),
    ],
)
def _deg_kernel(dst_hbm, out_hbm, idx_v, ones_v, zb_v, acc_sh):
    c = lax.axis_index("c")
    s = lax.axis_index("s")
    wid = s * NC + c
    pltpu.sync_copy(dst_hbm.at[wid], idx_v)
    for k in range(CH // 16):
        ones_v[pl.ds(16 * k, 16)] = jnp.ones((16,), jnp.float32)
    for k in range(RPT // 16):
        zb_v[pl.ds(16 * k, 16)] = jnp.zeros((16,), jnp.float32)
    pltpu.sync_copy(zb_v, acc_sh.at[pl.ds(s * RPT, RPT)])
    plsc.subcore_barrier()

    def body(j, carry):
        pltpu.sync_copy(ones_v, acc_sh.at[idx_v.at[j]], add=True)
        return carry

    lax.fori_loop(0, NCHK, body, 0)
    plsc.subcore_barrier()
    pltpu.sync_copy(acc_sh.at[pl.ds(s * RPT, RPT)],
                    out_hbm.at[pl.ds(c * NPAD + s * RPT, RPT)])


@functools.partial(
    pl.kernel,
    out_type=jax.ShapeDtypeStruct((NC * NPAD, H), jnp.float32),
    mesh=_mesh,
    compiler_params=pltpu.CompilerParams(use_tc_tiling_on_sc=False),
    scratch_types=[
        pltpu.VMEM((NCHK, CH), jnp.int32),
        pltpu.VMEM((NCHK, CH), jnp.int32),
        pltpu.VMEM((CH, H), jnp.float32),
        pltpu.VMEM((RPT, H), jnp.float32),
        pltpu.VMEM_SHARED((NPAD, H), jnp.float32),
        pltpu.SemaphoreType.DMA,
    ],
)
def _scat_kernel(g_hbm, src_hbm, dst_hbm, z_hbm, out_hbm,
                 src_v, dst_v, rows_v, zb_v, acc_sh, sem):
    c = lax.axis_index("c")
    s = lax.axis_index("s")
    wid = s * NC + c
    pltpu.sync_copy(src_hbm.at[wid], src_v)
    pltpu.sync_copy(dst_hbm.at[wid], dst_v)
    pltpu.sync_copy(z_hbm.at[pl.ds(s * RPT, RPT)], zb_v)
    pltpu.sync_copy(zb_v, acc_sh.at[pl.ds(s * RPT, RPT)])
    plsc.subcore_barrier()

    def body(j, carry):
        pltpu.async_copy(g_hbm.at[src_v.at[j]], rows_v, sem).wait()
        pltpu.sync_copy(rows_v, acc_sh.at[dst_v.at[j]], add=True)
        return carry

    lax.fori_loop(0, NCHK, body, 0)
    plsc.subcore_barrier()
    pltpu.sync_copy(acc_sh.at[pl.ds(s * RPT, RPT)],
                    out_hbm.at[pl.ds(c * NPAD + s * RPT, RPT)])


@functools.partial(
    pl.kernel,
    out_type=jax.ShapeDtypeStruct((E,), jnp.float32),
    mesh=_mesh,
    compiler_params=pltpu.CompilerParams(use_tc_tiling_on_sc=False),
    scratch_types=[
        pltpu.VMEM((NCHK, CH), jnp.int32),
        pltpu.VMEM((NCHK, CH), jnp.int32),
        pltpu.VMEM((CH,), jnp.float32),
        pltpu.VMEM((CH,), jnp.float32),
        pltpu.VMEM((CH,), jnp.float32),
        pltpu.SemaphoreType.DMA,
        pltpu.SemaphoreType.DMA,
    ],
)
def _edge_kernel(a_hbm, c_hbm, src_hbm, dst_hbm, out_hbm,
                 src_v, dst_v, va_v, vc_v, vo_v, sem_a, sem_c):
    c = lax.axis_index("c")
    s = lax.axis_index("s")
    wid = s * NC + c
    base = wid * EPT
    pltpu.sync_copy(src_hbm.at[wid], src_v)
    pltpu.sync_copy(dst_hbm.at[wid], dst_v)

    def body(j, carry):
        cp_a = pltpu.async_copy(a_hbm.at[src_v.at[j]], va_v, sem_a)
        cp_c = pltpu.async_copy(c_hbm.at[dst_v.at[j]], vc_v, sem_c)
        cp_a.wait()
        cp_c.wait()
        for k in range(CH // 16):
            sl = pl.ds(16 * k, 16)
            vo_v[sl] = va_v[sl] + vc_v[sl]
        pltpu.sync_copy(vo_v, out_hbm.at[pl.ds(base + j * CH, CH)])
        return carry

    lax.fori_loop(0, NCHK, body, 0)


def _tc_deg_body(degp_ref, dis_ref, invd_ref):
    deg = degp_ref[0, :] + degp_ref[1, :] + 1.0
    dis_ref[...] = lax.rsqrt(deg)
    invd_ref[...] = 1.0 / deg


_tc_deg = pl.pallas_call(
    _tc_deg_body,
    out_shape=[jax.ShapeDtypeStruct((NPAD,), jnp.float32),
               jax.ShapeDtypeStruct((NPAD,), jnp.float32)],
)


def _tc_a_body(x_ref, w1_ref, b1_ref, dis_ref, invd_ref, g0_ref, self1_ref):
    h0 = jnp.dot(x_ref[...], w1_ref[...], preferred_element_type=jnp.float32)
    g0_ref[...] = h0 * dis_ref[...]
    self1_ref[...] = h0 * invd_ref[...] + b1_ref[...]


_tc_a = pl.pallas_call(
    _tc_a_body,
    out_shape=[jax.ShapeDtypeStruct((N, H), jnp.float32),
               jax.ShapeDtypeStruct((N, H), jnp.float32)],
)


def _tc_b_body(s1_ref, self1_ref, dis_ref, invd_ref, w2_ref, b2_ref,
               g1_ref, self2_ref):
    ssum = s1_ref[0, :N, :] + s1_ref[1, :N, :]
    h1 = jnp.maximum(dis_ref[...] * ssum + self1_ref[...], 0.0)
    h1w = jnp.dot(h1, w2_ref[...], preferred_element_type=jnp.float32)
    g1_ref[...] = h1w * dis_ref[...]
    self2_ref[...] = h1w * invd_ref[...] + b2_ref[...]


_tc_b = pl.pallas_call(
    _tc_b_body,
    out_shape=[jax.ShapeDtypeStruct((N, H), jnp.float32),
               jax.ShapeDtypeStruct((N, H), jnp.float32)],
)


def _tc_c_body(s2_ref, self2_ref, dis_ref, wf_ref, bf_ref, a_ref, c_ref):
    ssum = s2_ref[0, :N, :] + s2_ref[1, :N, :]
    h2 = dis_ref[...] * ssum + self2_ref[...]
    a_ref[...] = jnp.dot(h2, wf_ref[:H, :],
                         preferred_element_type=jnp.float32) + bf_ref[...]
    c_ref[...] = jnp.dot(h2, wf_ref[H:, :],
                         preferred_element_type=jnp.float32)


_tc_c = pl.pallas_call(
    _tc_c_body,
    out_shape=[jax.ShapeDtypeStruct((N, 1), jnp.float32),
               jax.ShapeDtypeStruct((N, 1), jnp.float32)],
)


def kernel(x, edge_index, W1, b1, W2, b2, Wf, bf):
    src3 = edge_index[0].reshape(NW, NCHK, CH)
    dst3 = edge_index[1].reshape(NW, NCHK, CH)
    z16 = jnp.zeros((NPAD, H), jnp.float32)

    degp = _deg_kernel(dst3).reshape(NC, NPAD)
    dis_pad, invd_pad = _tc_deg(degp)
    dis = dis_pad[:N].reshape(N, 1)
    invd = invd_pad[:N].reshape(N, 1)

    g0, self1 = _tc_a(x, W1, b1, dis, invd)
    s1 = _scat_kernel(g0, src3, dst3, z16).reshape(NC, NPAD, H)
    g1, self2 = _tc_b(s1, self1, dis, invd, W2, b2)
    s2 = _scat_kernel(g1, src3, dst3, z16).reshape(NC, NPAD, H)
    a2, c2 = _tc_c(s2, self2, dis, Wf, bf)

    return _edge_kernel(a2.reshape(N), c2.reshape(N), src3, dst3)


# speedup vs baseline: 45.4666x; 1.8994x over previous
import functools

import jax
import jax.numpy as jnp
from jax import lax
from jax.experimental import pallas as pl
from jax.experimental.pallas import tpu as pltpu
from jax.experimental.pallas import tpu_sc as plsc

N = 10000
E = 320000
H = 16
F_IN = 128

NC = 2
NS = 16
NW = NC * NS
EPT = E // NW
CH = 100
NCHK = EPT // CH
KB = 10
NG = NCHK // (2 * KB)
CHE = 80
NCHE = EPT // CHE
KE = 5
NGE = NCHE // KE
NPAD = 10240
RPT = NPAD // NS

_mesh = plsc.VectorSubcoreMesh(core_axis_name="c", subcore_axis_name="s")


@functools.partial(
    pl.kernel,
    out_type=jax.ShapeDtypeStruct((NC * NPAD,), jnp.float32),
    mesh=_mesh,
    compiler_params=pltpu.CompilerParams(use_tc_tiling_on_sc=False),
    scratch_types=[
        pltpu.VMEM((NCHE, CHE), jnp.int32),
        pltpu.VMEM((CHE,), jnp.float32),
        pltpu.VMEM((RPT,), jnp.float32),
        pltpu.VMEM_SHARED((NPAD,), jnp.float32),
    ],
)
def _deg_kernel(dst_hbm, out_hbm, idx_v, ones_v, zb_v, acc_sh):
    c = lax.axis_index("c")
    s = lax.axis_index("s")
    wid = s * NC + c
    pltpu.sync_copy(dst_hbm.at[wid], idx_v)
    for k in range(CHE // 16):
        ones_v[pl.ds(16 * k, 16)] = jnp.ones((16,), jnp.float32)
    for k in range(RPT // 16):
        zb_v[pl.ds(16 * k, 16)] = jnp.zeros((16,), jnp.float32)
    pltpu.sync_copy(zb_v, acc_sh.at[pl.ds(s * RPT, RPT)])
    plsc.subcore_barrier()

    def body(j, carry):
        pltpu.sync_copy(ones_v, acc_sh.at[idx_v.at[j]], add=True)
        return carry

    lax.fori_loop(0, NCHE, body, 0)
    plsc.subcore_barrier()
    pltpu.sync_copy(acc_sh.at[pl.ds(s * RPT, RPT)],
                    out_hbm.at[pl.ds(c * NPAD + s * RPT, RPT)])


@functools.partial(
    pl.kernel,
    out_type=jax.ShapeDtypeStruct((NC * NPAD, H), jnp.float32),
    mesh=_mesh,
    compiler_params=pltpu.CompilerParams(use_tc_tiling_on_sc=False),
    scratch_types=[
        pltpu.VMEM((NCHK, CH), jnp.int32),
        pltpu.VMEM((NCHK, CH), jnp.int32),
        pltpu.VMEM((KB, CH, H), jnp.float32),
        pltpu.VMEM((KB, CH, H), jnp.float32),
        pltpu.VMEM((RPT, H), jnp.float32),
        pltpu.VMEM_SHARED((NPAD, H), jnp.float32),
        pltpu.SemaphoreType.DMA,
        pltpu.SemaphoreType.DMA,
        pltpu.SemaphoreType.DMA,
        pltpu.SemaphoreType.DMA,
    ],
)
def _scat_kernel(g_hbm, src_hbm, dst_hbm, z_hbm, out_hbm,
                 src_v, dst_v, ra_v, rb_v, zb_v, acc_sh,
                 sga, sgb, ssa, ssb):
    c = lax.axis_index("c")
    s = lax.axis_index("s")
    wid = s * NC + c
    pltpu.sync_copy(src_hbm.at[wid], src_v)
    pltpu.sync_copy(dst_hbm.at[wid], dst_v)
    pltpu.sync_copy(z_hbm.at[pl.ds(s * RPT, RPT)], zb_v)
    pltpu.sync_copy(zb_v, acc_sh.at[pl.ds(s * RPT, RPT)])
    plsc.subcore_barrier()

    def body(g, carry):
        j0 = g * (2 * KB)
        cga = [pltpu.async_copy(g_hbm.at[src_v.at[j0 + b]], ra_v.at[b], sga)
               for b in range(KB)]
        cgb = [pltpu.async_copy(g_hbm.at[src_v.at[j0 + KB + b]], rb_v.at[b], sgb)
               for b in range(KB)]
        for cp in cga:
            cp.wait()
        csa = [pltpu.async_copy(ra_v.at[b], acc_sh.at[dst_v.at[j0 + b]],
                                ssa, add=True)
               for b in range(KB)]
        for cp in cgb:
            cp.wait()
        csb = [pltpu.async_copy(rb_v.at[b], acc_sh.at[dst_v.at[j0 + KB + b]],
                                ssb, add=True)
               for b in range(KB)]
        for cp in csa:
            cp.wait()
        for cp in csb:
            cp.wait()
        return carry

    lax.fori_loop(0, NG, body, 0)
    plsc.subcore_barrier()
    pltpu.sync_copy(acc_sh.at[pl.ds(s * RPT, RPT)],
                    out_hbm.at[pl.ds(c * NPAD + s * RPT, RPT)])


@functools.partial(
    pl.kernel,
    out_type=jax.ShapeDtypeStruct((E,), jnp.float32),
    mesh=_mesh,
    compiler_params=pltpu.CompilerParams(use_tc_tiling_on_sc=False),
    scratch_types=[
        pltpu.VMEM((NCHE, CHE), jnp.int32),
        pltpu.VMEM((NCHE, CHE), jnp.int32),
        pltpu.VMEM((KE, CHE), jnp.float32),
        pltpu.VMEM((KE, CHE), jnp.float32),
        pltpu.VMEM((KE, CHE), jnp.float32),
        pltpu.SemaphoreType.DMA,
        pltpu.SemaphoreType.DMA,
        pltpu.SemaphoreType.DMA,
    ],
)
def _edge_kernel(a_hbm, c_hbm, src_hbm, dst_hbm, out_hbm,
                 src_v, dst_v, va_v, vc_v, vo_v, sem_a, sem_c, sem_o):
    c = lax.axis_index("c")
    s = lax.axis_index("s")
    wid = s * NC + c
    base = wid * EPT
    pltpu.sync_copy(src_hbm.at[wid], src_v)
    pltpu.sync_copy(dst_hbm.at[wid], dst_v)

    def body(g, carry):
        j0 = g * KE
        ca = [pltpu.async_copy(a_hbm.at[src_v.at[j0 + b]], va_v.at[b], sem_a)
              for b in range(KE)]
        cc = [pltpu.async_copy(c_hbm.at[dst_v.at[j0 + b]], vc_v.at[b], sem_c)
              for b in range(KE)]
        co = []
        for b in range(KE):
            ca[b].wait()
            cc[b].wait()
            for k in range(CHE // 16):
                sl = pl.ds(16 * k, 16)
                vo_v[b, sl] = va_v[b, sl] + vc_v[b, sl]
            co.append(pltpu.async_copy(
                vo_v.at[b], out_hbm.at[pl.ds(base + (j0 + b) * CHE, CHE)],
                sem_o))
        for cp in co:
            cp.wait()
        return carry

    lax.fori_loop(0, NGE, body, 0)


def _tc_deg_body(degp_ref, dis_ref, invd_ref):
    deg = degp_ref[0, :] + degp_ref[1, :] + 1.0
    dis_ref[...] = lax.rsqrt(deg)
    invd_ref[...] = 1.0 / deg


_tc_deg = pl.pallas_call(
    _tc_deg_body,
    out_shape=[jax.ShapeDtypeStruct((NPAD,), jnp.float32),
               jax.ShapeDtypeStruct((NPAD,), jnp.float32)],
)


def _tc_a_body(x_ref, w1_ref, b1_ref, dis_ref, invd_ref, g0_ref, self1_ref):
    h0 = jnp.dot(x_ref[...], w1_ref[...], preferred_element_type=jnp.float32)
    g0_ref[...] = h0 * dis_ref[...]
    self1_ref[...] = h0 * invd_ref[...] + b1_ref[...]


_tc_a = pl.pallas_call(
    _tc_a_body,
    out_shape=[jax.ShapeDtypeStruct((N, H), jnp.float32),
               jax.ShapeDtypeStruct((N, H), jnp.float32)],
)


def _tc_b_body(s1_ref, self1_ref, dis_ref, invd_ref, w2_ref, b2_ref,
               g1_ref, self2_ref):
    ssum = s1_ref[0, :N, :] + s1_ref[1, :N, :]
    h1 = jnp.maximum(dis_ref[...] * ssum + self1_ref[...], 0.0)
    h1w = jnp.dot(h1, w2_ref[...], preferred_element_type=jnp.float32)
    g1_ref[...] = h1w * dis_ref[...]
    self2_ref[...] = h1w * invd_ref[...] + b2_ref[...]


_tc_b = pl.pallas_call(
    _tc_b_body,
    out_shape=[jax.ShapeDtypeStruct((N, H), jnp.float32),
               jax.ShapeDtypeStruct((N, H), jnp.float32)],
)


def _tc_c_body(s2_ref, self2_ref, dis_ref, wf_ref, bf_ref, a_ref, c_ref):
    ssum = s2_ref[0, :N, :] + s2_ref[1, :N, :]
    h2 = dis_ref[...] * ssum + self2_ref[...]
    a_ref[...] = jnp.dot(h2, wf_ref[:H, :],
                         preferred_element_type=jnp.float32) + bf_ref[...]
    c_ref[...] = jnp.dot(h2, wf_ref[H:, :],
                         preferred_element_type=jnp.float32)


_tc_c = pl.pallas_call(
    _tc_c_body,
    out_shape=[jax.ShapeDtypeStruct((N, 1), jnp.float32),
               jax.ShapeDtypeStruct((N, 1), jnp.float32)],
)


def kernel(x, edge_index, W1, b1, W2, b2, Wf, bf):
    src3 = edge_index[0].reshape(NW, NCHK, CH)
    dst3 = edge_index[1].reshape(NW, NCHK, CH)
    src3e = edge_index[0].reshape(NW, NCHE, CHE)
    dst3e = edge_index[1].reshape(NW, NCHE, CHE)
    z16 = jnp.zeros((NPAD, H), jnp.float32)

    degp = _deg_kernel(dst3e).reshape(NC, NPAD)
    dis_pad, invd_pad = _tc_deg(degp)
    dis = dis_pad[:N].reshape(N, 1)
    invd = invd_pad[:N].reshape(N, 1)

    g0, self1 = _tc_a(x, W1, b1, dis, invd)
    s1 = _scat_kernel(g0, src3, dst3, z16).reshape(NC, NPAD, H)
    g1, self2 = _tc_b(s1, self1, dis, invd, W2, b2)
    s2 = _scat_kernel(g1, src3, dst3, z16).reshape(NC, NPAD, H)
    a2, c2 = _tc_c(s2, self2, dis, Wf, bf)

    return _edge_kernel(a2.reshape(N), c2.reshape(N), src3e, dst3e)


# speedup vs baseline: 65.5863x; 1.4425x over previous
import functools

import jax
import jax.numpy as jnp
from jax import lax
from jax.experimental import pallas as pl
from jax.experimental.pallas import tpu as pltpu
from jax.experimental.pallas import tpu_sc as plsc

N = 10000
E = 320000
H = 16
F_IN = 128

NC = 2
NS = 16
NW = NC * NS
EPT = E // NW
CHE = 80
NCHE = EPT // CHE
KB = 10
NG = NCHE // (2 * KB)
TAIL = NCHE - NG * 2 * KB
KD = 25
KE = 25
NPAD = 10240
RPT = NPAD // NS
N8 = N // 8
NP8 = NPAD // 8

_mesh = plsc.VectorSubcoreMesh(core_axis_name="c", subcore_axis_name="s")


@functools.partial(
    pl.kernel,
    out_type=jax.ShapeDtypeStruct((NC * NPAD * H,), jnp.float32),
    mesh=_mesh,
    compiler_params=pltpu.CompilerParams(use_tc_tiling_on_sc=False),
    scratch_types=[
        pltpu.VMEM((NCHE, CHE), jnp.int32),
        pltpu.VMEM((CHE,), jnp.float32),
        pltpu.VMEM((RPT,), jnp.float32),
        pltpu.VMEM((RPT * H,), jnp.float32),
        pltpu.VMEM_SHARED((NPAD,), jnp.float32),
        pltpu.SemaphoreType.DMA,
    ],
)
def _deg_kernel(dst_hbm, out_hbm, idx_v, ones_v, accv_v, exp_v, acc_sh, sem):
    c = lax.axis_index("c")
    s = lax.axis_index("s")
    wid = s * NC + c
    pltpu.sync_copy(dst_hbm.at[wid], idx_v)
    for k in range(CHE // 16):
        ones_v[pl.ds(16 * k, 16)] = jnp.ones((16,), jnp.float32)
    for k in range(RPT // 16):
        accv_v[pl.ds(16 * k, 16)] = jnp.zeros((16,), jnp.float32)
    pltpu.sync_copy(accv_v, acc_sh.at[pl.ds(s * RPT, RPT)])
    plsc.subcore_barrier()

    def body(g, carry):
        j0 = g * KD
        cps = [pltpu.async_copy(ones_v, acc_sh.at[idx_v.at[j0 + b]],
                                sem, add=True)
               for b in range(KD)]
        for cp in cps:
            cp.wait()
        return carry

    lax.fori_loop(0, NCHE // KD, body, 0)
    plsc.subcore_barrier()
    pltpu.sync_copy(acc_sh.at[pl.ds(s * RPT, RPT)], accv_v)

    def expand(i, carry):
        v = accv_v[pl.ds(16 * i, 16)]
        for j in range(16):
            exp_v[pl.ds(H * (16 * i + j), H)] = jnp.broadcast_to(v[j], (H,))
        return carry

    lax.fori_loop(0, RPT // 16, expand, 0)
    pltpu.sync_copy(exp_v,
                    out_hbm.at[pl.ds((c * NPAD + s * RPT) * H, RPT * H)])


@functools.partial(
    pl.kernel,
    out_type=jax.ShapeDtypeStruct((NC * NPAD, H), jnp.float32),
    mesh=_mesh,
    compiler_params=pltpu.CompilerParams(use_tc_tiling_on_sc=False),
    scratch_types=[
        pltpu.VMEM((NCHE, CHE), jnp.int32),
        pltpu.VMEM((NCHE, CHE), jnp.int32),
        pltpu.VMEM((KB, CHE, H), jnp.float32),
        pltpu.VMEM((KB, CHE, H), jnp.float32),
        pltpu.VMEM((RPT, H), jnp.float32),
        pltpu.VMEM_SHARED((NPAD, H), jnp.float32),
        pltpu.SemaphoreType.DMA,
        pltpu.SemaphoreType.DMA,
        pltpu.SemaphoreType.DMA,
        pltpu.SemaphoreType.DMA,
    ],
)
def _scat_kernel(g_hbm, src_hbm, dst_hbm, z_hbm, out_hbm,
                 src_v, dst_v, ra_v, rb_v, zb_v, acc_sh,
                 sga, sgb, ssa, ssb):
    c = lax.axis_index("c")
    s = lax.axis_index("s")
    wid = s * NC + c
    pltpu.sync_copy(src_hbm.at[wid], src_v)
    pltpu.sync_copy(dst_hbm.at[wid], dst_v)
    pltpu.sync_copy(z_hbm.at[pl.ds(s * RPT, RPT)], zb_v)
    pltpu.sync_copy(zb_v, acc_sh.at[pl.ds(s * RPT, RPT)])
    plsc.subcore_barrier()

    def body(g, carry):
        j0 = g * (2 * KB)
        cga = [pltpu.async_copy(g_hbm.at[src_v.at[j0 + b]], ra_v.at[b], sga)
               for b in range(KB)]
        cgb = [pltpu.async_copy(g_hbm.at[src_v.at[j0 + KB + b]], rb_v.at[b],
                                sgb)
               for b in range(KB)]
        for cp in cga:
            cp.wait()
        csa = [pltpu.async_copy(ra_v.at[b], acc_sh.at[dst_v.at[j0 + b]],
                                ssa, add=True)
               for b in range(KB)]
        for cp in cgb:
            cp.wait()
        csb = [pltpu.async_copy(rb_v.at[b], acc_sh.at[dst_v.at[j0 + KB + b]],
                                ssb, add=True)
               for b in range(KB)]
        for cp in csa:
            cp.wait()
        for cp in csb:
            cp.wait()
        return carry

    lax.fori_loop(0, NG, body, 0)

    t0 = NG * 2 * KB
    ct = [pltpu.async_copy(g_hbm.at[src_v.at[t0 + b]], ra_v.at[b], sga)
          for b in range(TAIL)]
    for cp in ct:
        cp.wait()
    cs = [pltpu.async_copy(ra_v.at[b], acc_sh.at[dst_v.at[t0 + b]],
                           ssa, add=True)
          for b in range(TAIL)]
    for cp in cs:
        cp.wait()

    plsc.subcore_barrier()
    pltpu.sync_copy(acc_sh.at[pl.ds(s * RPT, RPT)],
                    out_hbm.at[pl.ds(c * NPAD + s * RPT, RPT)])


@functools.partial(
    pl.kernel,
    out_type=jax.ShapeDtypeStruct((E,), jnp.float32),
    mesh=_mesh,
    compiler_params=pltpu.CompilerParams(use_tc_tiling_on_sc=False),
    scratch_types=[
        pltpu.VMEM((NCHE, CHE), jnp.int32),
        pltpu.VMEM((NCHE, CHE), jnp.int32),
        pltpu.VMEM((KE, CHE), jnp.float32),
        pltpu.VMEM((KE, CHE), jnp.float32),
        pltpu.VMEM((KE, CHE), jnp.float32),
        pltpu.SemaphoreType.DMA,
        pltpu.SemaphoreType.DMA,
        pltpu.SemaphoreType.DMA,
    ],
)
def _edge_kernel(a_hbm, c_hbm, src_hbm, dst_hbm, out_hbm,
                 src_v, dst_v, va_v, vc_v, vo_v, sem_a, sem_c, sem_o):
    c = lax.axis_index("c")
    s = lax.axis_index("s")
    wid = s * NC + c
    base = wid * EPT
    pltpu.sync_copy(src_hbm.at[wid], src_v)
    pltpu.sync_copy(dst_hbm.at[wid], dst_v)

    def body(g, carry):
        j0 = g * KE
        ca = [pltpu.async_copy(a_hbm.at[src_v.at[j0 + b]], va_v.at[b], sem_a)
              for b in range(KE)]
        cc = [pltpu.async_copy(c_hbm.at[dst_v.at[j0 + b]], vc_v.at[b], sem_c)
              for b in range(KE)]
        co = []
        for b in range(KE):
            ca[b].wait()
            cc[b].wait()
            for k in range(CHE // 16):
                sl = pl.ds(16 * k, 16)
                vo_v[b, sl] = va_v[b, sl] + vc_v[b, sl]
            co.append(pltpu.async_copy(
                vo_v.at[b], out_hbm.at[pl.ds(base + (j0 + b) * CHE, CHE)],
                sem_o))
        for cp in co:
            cp.wait()
        return carry

    lax.fori_loop(0, NCHE // KE, body, 0)


def _tc_a_body(degp_ref, x3_ref, w1_ref, b1p_ref,
               g0_ref, self1_ref, dis_ref, invd_ref):
    deg = degp_ref[0] + degp_ref[1] + 1.0
    dis_p = lax.rsqrt(deg)
    invd_p = 1.0 / deg
    dis_ref[...] = dis_p
    invd_ref[...] = invd_p
    h0p = jnp.concatenate(
        [jnp.dot(x3_ref[:, k, :], w1_ref[...],
                 preferred_element_type=jnp.float32)
         for k in range(8)], axis=1)
    g0_ref[...] = h0p * dis_p[:N8]
    self1_ref[...] = h0p * invd_p[:N8] + b1p_ref[...]


_tc_a = pl.pallas_call(
    _tc_a_body,
    out_shape=[jax.ShapeDtypeStruct((N8, 128), jnp.float32),
               jax.ShapeDtypeStruct((N8, 128), jnp.float32),
               jax.ShapeDtypeStruct((NP8, 128), jnp.float32),
               jax.ShapeDtypeStruct((NP8, 128), jnp.float32)],
)


def _tc_b_body(s1_ref, self1_ref, dis_ref, invd_ref, w2b_ref, b2p_ref,
               g1_ref, self2_ref):
    ssum = s1_ref[0, :N8, :] + s1_ref[1, :N8, :]
    h1p = jnp.maximum(dis_ref[:N8, :] * ssum + self1_ref[...], 0.0)
    h1wp = jnp.dot(h1p, w2b_ref[...], preferred_element_type=jnp.float32)
    g1_ref[...] = h1wp * dis_ref[:N8, :]
    self2_ref[...] = h1wp * invd_ref[:N8, :] + b2p_ref[...]


_tc_b = pl.pallas_call(
    _tc_b_body,
    out_shape=[jax.ShapeDtypeStruct((N8, 128), jnp.float32),
               jax.ShapeDtypeStruct((N8, 128), jnp.float32)],
)


def _tc_c_body(s2_ref, self2_ref, dis_ref, wfab_ref, bfv_ref, ac_ref):
    ssum = s2_ref[0, :N8, :] + s2_ref[1, :N8, :]
    h2p = dis_ref[:N8, :] * ssum + self2_ref[...]
    ac_ref[...] = jnp.dot(h2p, wfab_ref[...],
                          preferred_element_type=jnp.float32) + bfv_ref[...]


_tc_c = pl.pallas_call(
    _tc_c_body,
    out_shape=jax.ShapeDtypeStruct((N8, 16), jnp.float32),
)


def kernel(x, edge_index, W1, b1, W2, b2, Wf, bf):
    src3 = edge_index[0].reshape(NW, NCHE, CHE)
    dst3 = edge_index[1].reshape(NW, NCHE, CHE)
    z16 = jnp.zeros((NPAD, H), jnp.float32)
    x3 = x.reshape(N8, 8, F_IN)
    eye8 = jnp.eye(8, dtype=jnp.float32)
    w2b = jnp.kron(eye8, W2)
    wfab = jnp.concatenate([jnp.kron(eye8, Wf[:H]),
                            jnp.kron(eye8, Wf[H:])], axis=1)
    b1p = jnp.tile(b1, 8)
    b2p = jnp.tile(b2, 8)
    bfv = jnp.concatenate([jnp.broadcast_to(bf, (8,)),
                           jnp.zeros((8,), jnp.float32)])

    degp = _deg_kernel(dst3).reshape(NC, NP8, 128)
    g0p, self1p, disp, invdp = _tc_a(degp, x3, W1, b1p)
    s1 = _scat_kernel(g0p.reshape(N, H), src3, dst3, z16).reshape(NC, NP8, 128)
    g1p, self2p = _tc_b(s1, self1p, disp, invdp, w2b, b2p)
    s2 = _scat_kernel(g1p.reshape(N, H), src3, dst3, z16).reshape(NC, NP8, 128)
    ac = _tc_c(s2, self2p, disp, wfab, bfv)
    a2 = ac[:, 0:8].reshape(N)
    c2 = ac[:, 8:16].reshape(N)
    return _edge_kernel(a2, c2, src3, dst3)


# speedup vs baseline: 84.8451x; 1.2936x over previous
import functools

import jax
import jax.numpy as jnp
from jax import lax
from jax.experimental import pallas as pl
from jax.experimental.pallas import tpu as pltpu
from jax.experimental.pallas import tpu_sc as plsc

N = 10000
E = 320000
H = 16
F_IN = 128

NC = 2
NS = 16
NW = NC * NS
EPT = E // NW
CHE = 80
NCHE = EPT // CHE
KB = 10
NG = NCHE // (2 * KB)
TAIL = NCHE - NG * 2 * KB
KD = 25
KE = 25
NPAD = 10240
RPT = NPAD // NS
N8 = N // 8
NP8 = NPAD // 8

_mesh = plsc.VectorSubcoreMesh(core_axis_name="c", subcore_axis_name="s")


@functools.partial(
    pl.kernel,
    out_type=jax.ShapeDtypeStruct((NC * NPAD * H,), jnp.float32),
    mesh=_mesh,
    compiler_params=pltpu.CompilerParams(use_tc_tiling_on_sc=False),
    scratch_types=[
        pltpu.VMEM((NCHE, CHE), jnp.int32),
        pltpu.VMEM((CHE,), jnp.float32),
        pltpu.VMEM((RPT,), jnp.float32),
        pltpu.VMEM((RPT * H,), jnp.float32),
        pltpu.VMEM_SHARED((NPAD,), jnp.float32),
        pltpu.SemaphoreType.DMA,
    ],
)
def _deg_kernel(dst_hbm, out_hbm, idx_v, ones_v, accv_v, exp_v, acc_sh, sem):
    c = lax.axis_index("c")
    s = lax.axis_index("s")
    wid = s * NC + c
    pltpu.sync_copy(dst_hbm.at[wid], idx_v)
    for k in range(CHE // 16):
        ones_v[pl.ds(16 * k, 16)] = jnp.ones((16,), jnp.float32)
    for k in range(RPT // 16):
        accv_v[pl.ds(16 * k, 16)] = jnp.zeros((16,), jnp.float32)
    pltpu.sync_copy(accv_v, acc_sh.at[pl.ds(s * RPT, RPT)])
    plsc.subcore_barrier()

    def body(g, carry):
        j0 = g * KD
        cps = [pltpu.async_copy(ones_v, acc_sh.at[idx_v.at[j0 + b]],
                                sem, add=True)
               for b in range(KD)]
        for cp in cps:
            cp.wait()
        return carry

    lax.fori_loop(0, NCHE // KD, body, 0)
    plsc.subcore_barrier()
    pltpu.sync_copy(acc_sh.at[pl.ds(s * RPT, RPT)], accv_v)

    def expand(i, carry):
        v = accv_v[pl.ds(16 * i, 16)]
        for j in range(16):
            exp_v[pl.ds(H * (16 * i + j), H)] = jnp.broadcast_to(v[j], (H,))
        return carry

    lax.fori_loop(0, RPT // 16, expand, 0)
    pltpu.sync_copy(exp_v,
                    out_hbm.at[pl.ds((c * NPAD + s * RPT) * H, RPT * H)])


@functools.partial(
    pl.kernel,
    out_type=jax.ShapeDtypeStruct((NC * NPAD, H), jnp.float32),
    mesh=_mesh,
    compiler_params=pltpu.CompilerParams(use_tc_tiling_on_sc=False),
    scratch_types=[
        pltpu.VMEM((NCHE, CHE), jnp.int32),
        pltpu.VMEM((NCHE, CHE), jnp.int32),
        pltpu.VMEM((KB, CHE, H), jnp.float32),
        pltpu.VMEM((KB, CHE, H), jnp.float32),
        pltpu.VMEM((RPT, H), jnp.float32),
        pltpu.VMEM_SHARED((NPAD, H), jnp.float32),
        pltpu.SemaphoreType.DMA,
        pltpu.SemaphoreType.DMA,
        pltpu.SemaphoreType.DMA,
        pltpu.SemaphoreType.DMA,
    ],
)
def _scat_kernel(g_hbm, src_hbm, dst_hbm, z_hbm, out_hbm,
                 src_v, dst_v, ra_v, rb_v, zb_v, acc_sh,
                 sga, sgb, ssa, ssb):
    c = lax.axis_index("c")
    s = lax.axis_index("s")
    wid = s * NC + c
    pltpu.sync_copy(src_hbm.at[wid], src_v)
    pltpu.sync_copy(dst_hbm.at[wid], dst_v)
    pltpu.sync_copy(z_hbm.at[pl.ds(s * RPT, RPT)], zb_v)
    pltpu.sync_copy(zb_v, acc_sh.at[pl.ds(s * RPT, RPT)])
    plsc.subcore_barrier()

    def body(g, carry):
        j0 = g * (2 * KB)
        cga = [pltpu.async_copy(g_hbm.at[src_v.at[j0 + b]], ra_v.at[b], sga)
               for b in range(KB)]
        cgb = [pltpu.async_copy(g_hbm.at[src_v.at[j0 + KB + b]], rb_v.at[b],
                                sgb)
               for b in range(KB)]
        for cp in cga:
            cp.wait()
        csa = [pltpu.async_copy(ra_v.at[b], acc_sh.at[dst_v.at[j0 + b]],
                                ssa, add=True)
               for b in range(KB)]
        for cp in cgb:
            cp.wait()
        csb = [pltpu.async_copy(rb_v.at[b], acc_sh.at[dst_v.at[j0 + KB + b]],
                                ssb, add=True)
               for b in range(KB)]
        for cp in csa:
            cp.wait()
        for cp in csb:
            cp.wait()
        return carry

    lax.fori_loop(0, NG, body, 0)

    t0 = NG * 2 * KB
    ct = [pltpu.async_copy(g_hbm.at[src_v.at[t0 + b]], ra_v.at[b], sga)
          for b in range(TAIL)]
    for cp in ct:
        cp.wait()
    cs = [pltpu.async_copy(ra_v.at[b], acc_sh.at[dst_v.at[t0 + b]],
                           ssa, add=True)
          for b in range(TAIL)]
    for cp in cs:
        cp.wait()

    plsc.subcore_barrier()
    pltpu.sync_copy(acc_sh.at[pl.ds(s * RPT, RPT)],
                    out_hbm.at[pl.ds(c * NPAD + s * RPT, RPT)])


@functools.partial(
    pl.kernel,
    out_type=jax.ShapeDtypeStruct((E,), jnp.float32),
    mesh=_mesh,
    compiler_params=pltpu.CompilerParams(use_tc_tiling_on_sc=False,
                                         needs_layout_passes=False),
    scratch_types=[
        pltpu.VMEM((N,), jnp.float32),
        pltpu.VMEM((N,), jnp.float32),
        pltpu.VMEM((EPT,), jnp.int32),
        pltpu.VMEM((EPT,), jnp.int32),
        pltpu.VMEM((EPT,), jnp.float32),
    ],
)
def _edge_kernel(a_hbm, c_hbm, src_hbm, dst_hbm, out_hbm,
                 a_v, c_v, src_v, dst_v, vo_v):
    c = lax.axis_index("c")
    s = lax.axis_index("s")
    wid = s * NC + c
    pltpu.sync_copy(a_hbm, a_v)
    pltpu.sync_copy(c_hbm, c_v)
    pltpu.sync_copy(src_hbm.at[wid], src_v)
    pltpu.sync_copy(dst_hbm.at[wid], dst_v)

    def body(i, carry):
        sl = pl.ds(16 * i, 16)
        av = plsc.load_gather(a_v, [src_v[sl]])
        cv = plsc.load_gather(c_v, [dst_v[sl]])
        vo_v[sl] = av + cv
        return carry

    lax.fori_loop(0, EPT // 16, body, 0)
    pltpu.sync_copy(vo_v, out_hbm.at[pl.ds(wid * EPT, EPT)])


def _tc_a_body(degp_ref, x3_ref, w1_ref, b1p_ref,
               g0_ref, self1_ref, dis_ref, invd_ref):
    deg = degp_ref[0] + degp_ref[1] + 1.0
    dis_p = lax.rsqrt(deg)
    invd_p = 1.0 / deg
    dis_ref[...] = dis_p
    invd_ref[...] = invd_p
    h0p = jnp.concatenate(
        [jnp.dot(x3_ref[:, k, :], w1_ref[...],
                 preferred_element_type=jnp.float32)
         for k in range(8)], axis=1)
    g0_ref[...] = h0p * dis_p[:N8]
    self1_ref[...] = h0p * invd_p[:N8] + b1p_ref[...]


_tc_a = pl.pallas_call(
    _tc_a_body,
    out_shape=[jax.ShapeDtypeStruct((N8, 128), jnp.float32),
               jax.ShapeDtypeStruct((N8, 128), jnp.float32),
               jax.ShapeDtypeStruct((NP8, 128), jnp.float32),
               jax.ShapeDtypeStruct((NP8, 128), jnp.float32)],
)


def _tc_b_body(s1_ref, self1_ref, dis_ref, invd_ref, w2b_ref, b2p_ref,
               g1_ref, self2_ref):
    ssum = s1_ref[0, :N8, :] + s1_ref[1, :N8, :]
    h1p = jnp.maximum(dis_ref[:N8, :] * ssum + self1_ref[...], 0.0)
    h1wp = jnp.dot(h1p, w2b_ref[...], preferred_element_type=jnp.float32)
    g1_ref[...] = h1wp * dis_ref[:N8, :]
    self2_ref[...] = h1wp * invd_ref[:N8, :] + b2p_ref[...]


_tc_b = pl.pallas_call(
    _tc_b_body,
    out_shape=[jax.ShapeDtypeStruct((N8, 128), jnp.float32),
               jax.ShapeDtypeStruct((N8, 128), jnp.float32)],
)


def _tc_c_body(s2_ref, self2_ref, dis_ref, wfab_ref, bfv_ref, ac_ref):
    ssum = s2_ref[0, :N8, :] + s2_ref[1, :N8, :]
    h2p = dis_ref[:N8, :] * ssum + self2_ref[...]
    ac_ref[...] = jnp.dot(h2p, wfab_ref[...],
                          preferred_element_type=jnp.float32) + bfv_ref[...]


_tc_c = pl.pallas_call(
    _tc_c_body,
    out_shape=jax.ShapeDtypeStruct((N8, 16), jnp.float32),
)


def kernel(x, edge_index, W1, b1, W2, b2, Wf, bf):
    src3 = edge_index[0].reshape(NW, NCHE, CHE)
    dst3 = edge_index[1].reshape(NW, NCHE, CHE)
    z16 = jnp.zeros((NPAD, H), jnp.float32)
    x3 = x.reshape(N8, 8, F_IN)
    eye8 = jnp.eye(8, dtype=jnp.float32)
    w2b = jnp.kron(eye8, W2)
    wfab = jnp.concatenate([jnp.kron(eye8, Wf[:H]),
                            jnp.kron(eye8, Wf[H:])], axis=1)
    b1p = jnp.tile(b1, 8)
    b2p = jnp.tile(b2, 8)
    bfv = jnp.concatenate([jnp.broadcast_to(bf, (8,)),
                           jnp.zeros((8,), jnp.float32)])

    degp = _deg_kernel(dst3).reshape(NC, NP8, 128)
    g0p, self1p, disp, invdp = _tc_a(degp, x3, W1, b1p)
    s1 = _scat_kernel(g0p.reshape(N, H), src3, dst3, z16).reshape(NC, NP8, 128)
    g1p, self2p = _tc_b(s1, self1p, disp, invdp, w2b, b2p)
    s2 = _scat_kernel(g1p.reshape(N, H), src3, dst3, z16).reshape(NC, NP8, 128)
    ac = _tc_c(s2, self2p, disp, wfab, bfv)
    a2 = ac[:, 0:8].reshape(N)
    c2 = ac[:, 8:16].reshape(N)
    src2 = src3.reshape(NW, EPT)
    dst2 = dst3.reshape(NW, EPT)
    return _edge_kernel(a2, c2, src2, dst2)


# speedup vs baseline: 89.9467x; 1.0601x over previous
import functools

import jax
import jax.numpy as jnp
from jax import lax
from jax.experimental import pallas as pl
from jax.experimental.pallas import tpu as pltpu
from jax.experimental.pallas import tpu_sc as plsc

N = 10000
E = 320000
H = 16
F_IN = 128

NC = 2
NS = 16
NW = NC * NS
EPT = E // NW
CHE = 80
NCHE = EPT // CHE
KB = 10
NG = NCHE // (2 * KB)
TAIL = NCHE - NG * 2 * KB
KD = 25
KE = 25
NPAD = 10240
RPT = NPAD // NS
GRS = N // NS
N8 = N // 8
NP8 = NPAD // 8

_mesh = plsc.VectorSubcoreMesh(core_axis_name="c", subcore_axis_name="s")


@functools.partial(
    pl.kernel,
    out_type=jax.ShapeDtypeStruct((NC * NPAD * H,), jnp.float32),
    mesh=_mesh,
    compiler_params=pltpu.CompilerParams(use_tc_tiling_on_sc=False),
    scratch_types=[
        pltpu.VMEM((NCHE, CHE), jnp.int32),
        pltpu.VMEM((CHE,), jnp.float32),
        pltpu.VMEM((RPT,), jnp.float32),
        pltpu.VMEM((RPT * H,), jnp.float32),
        pltpu.VMEM_SHARED((NPAD,), jnp.float32),
        pltpu.SemaphoreType.DMA,
    ],
)
def _deg_kernel(dst_hbm, out_hbm, idx_v, ones_v, accv_v, exp_v, acc_sh, sem):
    c = lax.axis_index("c")
    s = lax.axis_index("s")
    wid = s * NC + c
    pltpu.sync_copy(dst_hbm.at[wid], idx_v)
    for k in range(CHE // 16):
        ones_v[pl.ds(16 * k, 16)] = jnp.ones((16,), jnp.float32)
    for k in range(RPT // 16):
        accv_v[pl.ds(16 * k, 16)] = jnp.zeros((16,), jnp.float32)
    pltpu.sync_copy(accv_v, acc_sh.at[pl.ds(s * RPT, RPT)])
    plsc.subcore_barrier()

    def body(g, carry):
        j0 = g * KD
        cps = [pltpu.async_copy(ones_v, acc_sh.at[idx_v.at[j0 + b]],
                                sem, add=True)
               for b in range(KD)]
        for cp in cps:
            cp.wait()
        return carry

    lax.fori_loop(0, NCHE // KD, body, 0)
    plsc.subcore_barrier()
    pltpu.sync_copy(acc_sh.at[pl.ds(s * RPT, RPT)], accv_v)

    def expand(i, carry):
        v = accv_v[pl.ds(16 * i, 16)]
        for j in range(16):
            exp_v[pl.ds(H * (16 * i + j), H)] = jnp.broadcast_to(v[j], (H,))
        return carry

    lax.fori_loop(0, RPT // 16, expand, 0)
    pltpu.sync_copy(exp_v,
                    out_hbm.at[pl.ds((c * NPAD + s * RPT) * H, RPT * H)])


@functools.partial(
    pl.kernel,
    out_type=jax.ShapeDtypeStruct((NC * NPAD, H), jnp.float32),
    mesh=_mesh,
    compiler_params=pltpu.CompilerParams(use_tc_tiling_on_sc=False),
    scratch_types=[
        pltpu.VMEM((NCHE, CHE), jnp.int32),
        pltpu.VMEM((NCHE, CHE), jnp.int32),
        pltpu.VMEM((KB, CHE, H), jnp.float32),
        pltpu.VMEM((KB, CHE, H), jnp.float32),
        pltpu.VMEM((RPT, H), jnp.float32),
        pltpu.VMEM_SHARED((NPAD, H), jnp.float32),
        pltpu.VMEM_SHARED((N, H), jnp.float32),
        pltpu.SemaphoreType.DMA,
        pltpu.SemaphoreType.DMA,
        pltpu.SemaphoreType.DMA,
        pltpu.SemaphoreType.DMA,
    ],
)
def _scat_kernel(g_hbm, src_hbm, dst_hbm, z_hbm, out_hbm,
                 src_v, dst_v, ra_v, rb_v, zb_v, acc_sh, g_sh,
                 sga, sgb, ssa, ssb):
    c = lax.axis_index("c")
    s = lax.axis_index("s")
    wid = s * NC + c
    pltpu.sync_copy(src_hbm.at[wid], src_v)
    pltpu.sync_copy(dst_hbm.at[wid], dst_v)
    pltpu.sync_copy(g_hbm.at[pl.ds(s * GRS, GRS)], zb_v.at[pl.ds(0, GRS)])
    pltpu.sync_copy(zb_v.at[pl.ds(0, GRS)], g_sh.at[pl.ds(s * GRS, GRS)])
    pltpu.sync_copy(z_hbm.at[pl.ds(s * RPT, RPT)], zb_v)
    pltpu.sync_copy(zb_v, acc_sh.at[pl.ds(s * RPT, RPT)])
    plsc.subcore_barrier()

    def body(g, carry):
        j0 = g * (2 * KB)
        cga = [pltpu.async_copy(g_sh.at[src_v.at[j0 + b]], ra_v.at[b], sga)
               for b in range(KB)]
        cgb = [pltpu.async_copy(g_sh.at[src_v.at[j0 + KB + b]], rb_v.at[b],
                                sgb)
               for b in range(KB)]
        for cp in cga:
            cp.wait()
        csa = [pltpu.async_copy(ra_v.at[b], acc_sh.at[dst_v.at[j0 + b]],
                                ssa, add=True)
               for b in range(KB)]
        for cp in cgb:
            cp.wait()
        csb = [pltpu.async_copy(rb_v.at[b], acc_sh.at[dst_v.at[j0 + KB + b]],
                                ssb, add=True)
               for b in range(KB)]
        for cp in csa:
            cp.wait()
        for cp in csb:
            cp.wait()
        return carry

    lax.fori_loop(0, NG, body, 0)

    t0 = NG * 2 * KB
    ct = [pltpu.async_copy(g_sh.at[src_v.at[t0 + b]], ra_v.at[b], sga)
          for b in range(TAIL)]
    for cp in ct:
        cp.wait()
    cs = [pltpu.async_copy(ra_v.at[b], acc_sh.at[dst_v.at[t0 + b]],
                           ssa, add=True)
          for b in range(TAIL)]
    for cp in cs:
        cp.wait()

    plsc.subcore_barrier()
    pltpu.sync_copy(acc_sh.at[pl.ds(s * RPT, RPT)],
                    out_hbm.at[pl.ds(c * NPAD + s * RPT, RPT)])


@functools.partial(
    pl.kernel,
    out_type=jax.ShapeDtypeStruct((E,), jnp.float32),
    mesh=_mesh,
    compiler_params=pltpu.CompilerParams(use_tc_tiling_on_sc=False,
                                         needs_layout_passes=False),
    scratch_types=[
        pltpu.VMEM((N,), jnp.float32),
        pltpu.VMEM((N,), jnp.float32),
        pltpu.VMEM((EPT,), jnp.int32),
        pltpu.VMEM((EPT,), jnp.int32),
        pltpu.VMEM((EPT,), jnp.float32),
    ],
)
def _edge_kernel(a_hbm, c_hbm, src_hbm, dst_hbm, out_hbm,
                 a_v, c_v, src_v, dst_v, vo_v):
    c = lax.axis_index("c")
    s = lax.axis_index("s")
    wid = s * NC + c
    pltpu.sync_copy(a_hbm, a_v)
    pltpu.sync_copy(c_hbm, c_v)
    pltpu.sync_copy(src_hbm.at[wid], src_v)
    pltpu.sync_copy(dst_hbm.at[wid], dst_v)

    def body(i, carry):
        sl = pl.ds(16 * i, 16)
        av = plsc.load_gather(a_v, [src_v[sl]])
        cv = plsc.load_gather(c_v, [dst_v[sl]])
        vo_v[sl] = av + cv
        return carry

    lax.fori_loop(0, EPT // 16, body, 0)
    pltpu.sync_copy(vo_v, out_hbm.at[pl.ds(wid * EPT, EPT)])


def _tc_a_body(degp_ref, x3_ref, w1_ref, b1p_ref,
               g0_ref, self1_ref, dis_ref, invd_ref):
    deg = degp_ref[0] + degp_ref[1] + 1.0
    dis_p = lax.rsqrt(deg)
    invd_p = 1.0 / deg
    dis_ref[...] = dis_p
    invd_ref[...] = invd_p
    h0p = jnp.concatenate(
        [jnp.dot(x3_ref[:, k, :], w1_ref[...],
                 preferred_element_type=jnp.float32)
         for k in range(8)], axis=1)
    g0_ref[...] = h0p * dis_p[:N8]
    self1_ref[...] = h0p * invd_p[:N8] + b1p_ref[...]


_tc_a = pl.pallas_call(
    _tc_a_body,
    out_shape=[jax.ShapeDtypeStruct((N8, 128), jnp.float32),
               jax.ShapeDtypeStruct((N8, 128), jnp.float32),
               jax.ShapeDtypeStruct((NP8, 128), jnp.float32),
               jax.ShapeDtypeStruct((NP8, 128), jnp.float32)],
)


def _tc_b_body(s1_ref, self1_ref, dis_ref, invd_ref, w2b_ref, b2p_ref,
               g1_ref, self2_ref):
    ssum = s1_ref[0, :N8, :] + s1_ref[1, :N8, :]
    h1p = jnp.maximum(dis_ref[:N8, :] * ssum + self1_ref[...], 0.0)
    h1wp = jnp.dot(h1p, w2b_ref[...], preferred_element_type=jnp.float32)
    g1_ref[...] = h1wp * dis_ref[:N8, :]
    self2_ref[...] = h1wp * invd_ref[:N8, :] + b2p_ref[...]


_tc_b = pl.pallas_call(
    _tc_b_body,
    out_shape=[jax.ShapeDtypeStruct((N8, 128), jnp.float32),
               jax.ShapeDtypeStruct((N8, 128), jnp.float32)],
)


def _tc_c_body(s2_ref, self2_ref, dis_ref, wfab_ref, bfv_ref, ac_ref):
    ssum = s2_ref[0, :N8, :] + s2_ref[1, :N8, :]
    h2p = dis_ref[:N8, :] * ssum + self2_ref[...]
    ac_ref[...] = jnp.dot(h2p, wfab_ref[...],
                          preferred_element_type=jnp.float32) + bfv_ref[...]


_tc_c = pl.pallas_call(
    _tc_c_body,
    out_shape=jax.ShapeDtypeStruct((N8, 16), jnp.float32),
)


def kernel(x, edge_index, W1, b1, W2, b2, Wf, bf):
    src3 = edge_index[0].reshape(NW, NCHE, CHE)
    dst3 = edge_index[1].reshape(NW, NCHE, CHE)
    z16 = jnp.zeros((NPAD, H), jnp.float32)
    x3 = x.reshape(N8, 8, F_IN)
    eye8 = jnp.eye(8, dtype=jnp.float32)
    w2b = jnp.kron(eye8, W2)
    wfab = jnp.concatenate([jnp.kron(eye8, Wf[:H]),
                            jnp.kron(eye8, Wf[H:])], axis=1)
    b1p = jnp.tile(b1, 8)
    b2p = jnp.tile(b2, 8)
    bfv = jnp.concatenate([jnp.broadcast_to(bf, (8,)),
                           jnp.zeros((8,), jnp.float32)])

    degp = _deg_kernel(dst3).reshape(NC, NP8, 128)
    g0p, self1p, disp, invdp = _tc_a(degp, x3, W1, b1p)
    s1 = _scat_kernel(g0p.reshape(N, H), src3, dst3, z16).reshape(NC, NP8, 128)
    g1p, self2p = _tc_b(s1, self1p, disp, invdp, w2b, b2p)
    s2 = _scat_kernel(g1p.reshape(N, H), src3, dst3, z16).reshape(NC, NP8, 128)
    ac = _tc_c(s2, self2p, disp, wfab, bfv)
    a2 = ac[:, 0:8].reshape(N)
    c2 = ac[:, 8:16].reshape(N)
    src2 = src3.reshape(NW, EPT)
    dst2 = dst3.reshape(NW, EPT)
    return _edge_kernel(a2, c2, src2, dst2)
